# baseline (device time: 53813 ns/iter reference)
import jax
import jax.numpy as jnp
from jax import lax
from jax.experimental import pallas as pl
from jax.experimental.pallas import tpu as pltpu

N_DEV = 4
N_STEP = N_DEV - 1
N_Q = 4
N_RH = 2
N_LANE = N_Q * N_RH


def _gelu(z):
    return 0.5 * z * (1.0 + jnp.tanh(0.7978845608 * (z + 0.044715 * z * z * z)))


def kernel(A, B):
    m, k_per = A.shape
    _, n = B.shape
    m_chunk = m // N_DEV
    m_sub = m_chunk // N_RH
    n_q = n // N_Q

    def body(a_ref, b_ref, out_ref, a16, b16, partial_ref,
             srs, rsb, gsb, agb, rs_send, rs_recv, ag_send, ag_recv):
        my = lax.axis_index("i")
        left = lax.rem(my + N_DEV - 1, N_DEV)
        right = lax.rem(my + 1, N_DEV)

        def mod4(x):
            return lax.rem(x + 2 * N_DEV, N_DEV)

        lanes = [(q, 1 if q < 2 else -1, rh)
                 for rh in range(N_RH) for q in (0, 2, 1, 3)]

        def rows(c, lane):
            return pl.ds(c * m_chunk + lanes[lane][2] * m_sub, m_sub)

        def cq(lane):
            return pl.ds(lanes[lane][0] * n_q, n_q)

        def dev(lane):
            return right if lanes[lane][1] > 0 else left

        def c_recv(lane, s):
            return mod4(my - lanes[lane][1] * (s + 1))

        def mm(c, col_lo):
            a_chunk = a16[pl.ds(c * m_chunk, m_chunk), :]
            b_half = b16[:, 0:2 * n_q] if col_lo else b16[:, 2 * n_q:n]
            partial_ref[pl.ds(c * m_chunk, m_chunk),
                        pl.ds(0, 2 * n_q) if col_lo
                        else pl.ds(2 * n_q, 2 * n_q)] = jnp.dot(
                a_chunk, b_half, preferred_element_type=jnp.float32
            ).astype(jnp.bfloat16)

        a16[:, :] = a_ref[:, :].astype(jnp.bfloat16)
        b16[:, :] = b_ref[:, :].astype(jnp.bfloat16)

        barrier_sem = pltpu.get_barrier_semaphore()
        for nbr in (left, right):
            pl.semaphore_signal(
                barrier_sem, inc=1,
                device_id=(nbr,), device_id_type=pl.DeviceIdType.MESH,
            )
        pl.semaphore_wait(barrier_sem, 2)

        def sem_i(lane, s):
            return lane * N_STEP + s

        rs_d = [[pltpu.make_async_remote_copy(
                    src_ref=(partial_ref.at[rows(my, k), cq(k)] if s == 0
                             else srs.at[k, s - 1]),
                    dst_ref=rsb.at[k, s],
                    send_sem=rs_send.at[sem_i(k, s)],
                    recv_sem=rs_recv.at[sem_i(k, s)],
                    device_id=(dev(k),), device_id_type=pl.DeviceIdType.MESH)
                 for s in range(N_STEP)] for k in range(N_LANE)]
        ag_d = [[pltpu.make_async_remote_copy(
                    src_ref=(gsb.at[k] if h == 0 else agb.at[k, h - 1]),
                    dst_ref=agb.at[k, h],
                    send_sem=ag_send.at[sem_i(k, h)],
                    recv_sem=ag_recv.at[sem_i(k, h)],
                    device_id=(dev(k),), device_id_type=pl.DeviceIdType.MESH)
                 for h in range(N_STEP)] for k in range(N_LANE)]

        mm(my, True)
        mm(my, False)
        for k in range(N_LANE):
            rs_d[k][0].start()
        mm(mod4(my - 1), True)
        mm(mod4(my + 1), False)
        mm(mod4(my - 2), True)
        mm(mod4(my + 2), False)
        mm(mod4(my + 1), True)
        mm(mod4(my - 1), False)

        for s in range(N_STEP):
            for k in range(N_LANE):
                rs_d[k][s].wait()
                c = c_recv(k, s)
                if s < N_STEP - 1:
                    srs[k, s, :, :] = (
                        rsb[k, s, :, :] + partial_ref[rows(c, k), cq(k)]
                    )
                    rs_d[k][s + 1].start()
                else:
                    g = _gelu(rsb[k, s, :, :].astype(jnp.float32)
                              + partial_ref[rows(c, k), cq(k)].astype(jnp.float32))
                    gsb[k, :, :] = g.astype(jnp.bfloat16)
                    ag_d[k][0].start()
                    out_ref[rows(mod4(my + lanes[k][1]), k), cq(k)] = g

        for h in range(N_STEP):
            for k in range(N_LANE):
                ag_d[k][h].wait_recv()
                if h < N_STEP - 1:
                    ag_d[k][h + 1].start()
            for k in range(N_LANE):
                out_ref[rows(mod4(my - lanes[k][1] * h), k), cq(k)] = (
                    agb[k, h, :, :].astype(jnp.float32)
                )
        for k in range(N_LANE):
            for h in range(N_STEP):
                ag_d[k][h].wait_send()

    stage_shape = (N_LANE, N_STEP - 1, m_sub, n_q)
    lane_shape = (N_LANE, N_STEP, m_sub, n_q)
    n_sem = N_LANE * N_STEP
    return pl.pallas_call(
        body,
        out_shape=jax.ShapeDtypeStruct((m, n), jnp.float32),
        in_specs=[
            pl.BlockSpec(memory_space=pltpu.VMEM),
            pl.BlockSpec(memory_space=pltpu.VMEM),
        ],
        out_specs=pl.BlockSpec(memory_space=pltpu.VMEM),
        scratch_shapes=[
            pltpu.VMEM((m, k_per), jnp.bfloat16),
            pltpu.VMEM((k_per, n), jnp.bfloat16),
            pltpu.VMEM((m, n), jnp.bfloat16),
            pltpu.VMEM(stage_shape, jnp.bfloat16),
            pltpu.VMEM(lane_shape, jnp.bfloat16),
            pltpu.VMEM((N_LANE, m_sub, n_q), jnp.bfloat16),
            pltpu.VMEM(lane_shape, jnp.bfloat16),
            pltpu.SemaphoreType.DMA((n_sem,)),
            pltpu.SemaphoreType.DMA((n_sem,)),
            pltpu.SemaphoreType.DMA((n_sem,)),
            pltpu.SemaphoreType.DMA((n_sem,)),
        ],
        compiler_params=pltpu.CompilerParams(collective_id=0),
    )(A, B)
